# Initial kernel scaffold; baseline (speedup 1.0000x reference)
import jax
import jax.numpy as jnp
from jax.experimental import pallas as pl


def kernel(x, token_table, pos_table, seg_table, gamma, beta):
    raise NotImplementedError("write your pallas kernel here")


# speedup vs baseline: 7.4399x; 7.4399x over previous
import functools

import jax
import jax.numpy as jnp
from jax import lax
from jax.experimental import pallas as pl
from jax.experimental.pallas import tpu as pltpu
from jax.experimental.pallas import tpu_sc as plsc


def _sc_gather(table, idx_flat):
    n = idx_flat.shape[0]
    d = table.shape[1]
    info = plsc.get_sparse_core_info()
    nw = info.num_cores * info.num_subcores
    nc = info.num_cores
    per_w = n // nw
    ch = 192
    n_ch = per_w // ch
    assert per_w * nw == n and n_ch * ch == per_w

    mesh = plsc.VectorSubcoreMesh(core_axis_name="c", subcore_axis_name="s")

    @functools.partial(
        pl.kernel,
        mesh=mesh,
        out_type=jax.ShapeDtypeStruct((n, d), jnp.float32),
        scratch_types=[
            pltpu.VMEM((per_w,), jnp.int32),
            pltpu.VMEM((2, ch, d), jnp.float32),
            pltpu.SemaphoreType.DMA,
            pltpu.SemaphoreType.DMA,
        ],
    )
    def k(table_hbm, idx_hbm, out_hbm, idx_v, buf, gsem, ssem):
        wid = lax.axis_index("s") * nc + lax.axis_index("c")
        base = wid * per_w
        pltpu.sync_copy(idx_hbm.at[pl.ds(base, per_w)], idx_v)

        def step(c):
            slot = lax.rem(c, 2)
            row0 = c * ch
            pltpu.async_copy(
                table_hbm.at[idx_v.at[pl.ds(row0, ch)]], buf.at[slot], gsem
            ).wait()
            pltpu.async_copy(
                buf.at[slot], out_hbm.at[pl.ds(base + row0, ch)], ssem
            ).wait()

        pl.loop(0, n_ch)(step)

    return k(table, idx_flat)


def _tc_ln(tok, pos_table, seg_table, gamma, beta, max_sent):
    b, s, d = tok.shape
    bb = 8
    assert b % bb == 0

    def body(tok_ref, pos_ref, seg_ref, g_ref, b_ref, o_ref):
        h = tok_ref[...] + pos_ref[...][None, :, :]
        row = lax.broadcasted_iota(jnp.int32, (1, s, 1), 1)
        segv = jnp.where(row < max_sent + 1, seg_ref[0][None, None, :],
                         seg_ref[1][None, None, :])
        h = h + segv
        mean = jnp.mean(h, axis=-1, keepdims=True)
        c = h - mean
        var = jnp.mean(c * c, axis=-1, keepdims=True)
        o_ref[...] = (c * lax.rsqrt(var + 1e-5)) * g_ref[...] + b_ref[...]

    return pl.pallas_call(
        body,
        grid=(b // bb,),
        in_specs=[
            pl.BlockSpec((bb, s, d), lambda i: (i, 0, 0)),
            pl.BlockSpec((s, d), lambda i: (0, 0)),
            pl.BlockSpec((2, d), lambda i: (0, 0)),
            pl.BlockSpec((d,), lambda i: (0,)),
            pl.BlockSpec((d,), lambda i: (0,)),
        ],
        out_specs=pl.BlockSpec((bb, s, d), lambda i: (i, 0, 0)),
        out_shape=jax.ShapeDtypeStruct((b, s, d), jnp.float32),
    )(tok, pos_table, seg_table, gamma, beta)


def kernel(x, token_table, pos_table, seg_table, gamma, beta):
    b, s = x.shape
    d = token_table.shape[1]
    max_sent = (s - 2) // 2
    idx_flat = x.reshape(-1).astype(jnp.int32)
    tok = _sc_gather(token_table, idx_flat)
    return _tc_ln(tok.reshape(b, s, d), pos_table, seg_table, gamma, beta,
                  max_sent)
